# baseline (speedup 1.0000x reference)
import functools

import jax
import jax.numpy as jnp
from jax import lax
from jax.experimental import pallas as pl
from jax.experimental.pallas import tpu as pltpu
from jax.experimental.pallas import tpu_sc as plsc

B = 8
N = 4096
C = 128
M = 1024
K = 32
R2 = 0.2 * 0.2
EPS = 1e-5
S = B * M * K

NC, NS = 2, 16
NW = NC * NS

f32 = jnp.float32
i32 = jnp.int32


def _fps_body(xyz_ref, out_ref):
    X = xyz_ref[:, 0, :]
    Y = xyz_ref[:, 1, :]
    Z = xyz_ref[:, 2, :]
    iota = lax.broadcasted_iota(i32, (B, N), 1)

    def body(i, carry):
        dists, far = carry
        out_ref[pl.ds(i, 1), :] = far[None, :]
        oh = iota == far[:, None]
        cx = jnp.sum(jnp.where(oh, X, 0.0), axis=1, keepdims=True)
        cy = jnp.sum(jnp.where(oh, Y, 0.0), axis=1, keepdims=True)
        cz = jnp.sum(jnp.where(oh, Z, 0.0), axis=1, keepdims=True)
        dx = X - cx
        dy = Y - cy
        dz = Z - cz
        d = (dx * dx + dy * dy) + dz * dz
        dists = jnp.minimum(dists, d)
        far = jnp.argmax(dists, axis=1).astype(i32)
        return dists, far

    dists0 = jnp.full((B, N), 1e10, f32)
    far0 = jnp.zeros((B,), i32)
    lax.fori_loop(0, M, body, (dists0, far0))


def _fps(xyz):
    return pl.pallas_call(
        _fps_body,
        out_shape=jax.ShapeDtypeStruct((M, B), i32),
    )(xyz)


def _proj_body(f_ref, p_ref, wtf_ref, wtx_ref, wuf_ref, t_ref, u_ref):
    f = f_ref[0]
    p = p_ref[0]
    dn = (((0,), (0,)), ((), ()))
    xp = lax.dot_general(p, wtx_ref[...], dn, preferred_element_type=f32)
    t_ref[...] = lax.dot_general(f, wtf_ref[...], dn,
                                 preferred_element_type=f32) + xp
    u_ref[...] = lax.dot_general(f, wuf_ref[...], dn,
                                 preferred_element_type=f32) + xp


def _proj(feature, pts8, wtf, wtx, wuf):
    return pl.pallas_call(
        _proj_body,
        grid=(B,),
        in_specs=[
            pl.BlockSpec((1, C, N), lambda i: (i, 0, 0)),
            pl.BlockSpec((1, 8, N), lambda i: (i, 0, 0)),
            pl.BlockSpec((C, 128), lambda i: (0, 0)),
            pl.BlockSpec((8, 128), lambda i: (0, 0)),
            pl.BlockSpec((C, 128), lambda i: (0, 0)),
        ],
        out_specs=[
            pl.BlockSpec((N, 128), lambda i: (i, 0)),
            pl.BlockSpec((N, 128), lambda i: (i, 0)),
        ],
        out_shape=[
            jax.ShapeDtypeStruct((B * N, 128), f32),
            jax.ShapeDtypeStruct((B * N, 128), f32),
        ],
    )(feature, pts8, wtf, wtx, wuf)


def _sc_gather_small(table, idx):
    chn = idx.shape[1]
    rows = chn * 128
    mesh = plsc.VectorSubcoreMesh(core_axis_name="c", subcore_axis_name="s")

    @functools.partial(
        pl.kernel,
        mesh=mesh,
        out_type=jax.ShapeDtypeStruct((NW * rows, 128), f32),
        scratch_types=[
            pltpu.VMEM((chn, 128), i32),
            pltpu.VMEM((128, 128), f32),
            pltpu.SemaphoreType.DMA,
        ],
    )
    def k(table_hbm, idx_hbm, out_hbm, idx_v, rows_v, sem):
        wid = lax.axis_index("s") * NC + lax.axis_index("c")
        base = wid * rows
        pltpu.sync_copy(idx_hbm.at[wid], idx_v)

        def body(g, _):
            pltpu.async_copy(table_hbm.at[idx_v.at[g]], rows_v, sem).wait()
            pltpu.sync_copy(rows_v, out_hbm.at[pl.ds(base + g * 128, 128)])
            return 0

        lax.fori_loop(0, chn, body, 0)

    return k(table, idx)


def _sc_gather_big(ttab, utab, nbr_idx, fps_idx):
    chn = nbr_idx.shape[1]
    fch = fps_idx.shape[1]
    mesh = plsc.VectorSubcoreMesh(core_axis_name="c", subcore_axis_name="s")

    @functools.partial(
        pl.kernel,
        mesh=mesh,
        out_type=(
            jax.ShapeDtypeStruct((S, 128), f32),
            jax.ShapeDtypeStruct((B * M, 128), f32),
        ),
        scratch_types=[
            pltpu.VMEM((chn, 128), i32),
            pltpu.VMEM((128, 128), f32),
            pltpu.VMEM((128, 128), f32),
            pltpu.VMEM((128, 128), f32),
            pltpu.VMEM((128, 128), f32),
            pltpu.VMEM((fch, 128), i32),
            pltpu.SemaphoreType.DMA,
            pltpu.SemaphoreType.DMA,
            pltpu.SemaphoreType.DMA,
            pltpu.SemaphoreType.DMA,
            pltpu.SemaphoreType.DMA,
        ],
    )
    def k(t_hbm, u_hbm, nbr_hbm, fps_hbm, y1_hbm, c1_hbm,
          idx_v, r0, r1, r2, r3, fidx_v, s0, s1, s2, s3, sw):
        wid = lax.axis_index("s") * NC + lax.axis_index("c")
        base = wid * (chn * 128)
        pltpu.sync_copy(nbr_hbm.at[wid], idx_v)
        bufs = (r0, r1, r2, r3)
        sems = (s0, s1, s2, s3)

        def body(g4, _):
            g = g4 * 4
            hs = [pltpu.async_copy(t_hbm.at[idx_v.at[g + q]], bufs[q],
                                   sems[q]) for q in range(4)]
            ws = []
            for q in range(4):
                hs[q].wait()
                ws.append(pltpu.async_copy(
                    bufs[q], y1_hbm.at[pl.ds(base + (g + q) * 128, 128)], sw))
            for w in ws:
                w.wait()
            return 0

        lax.fori_loop(0, chn // 4, body, 0)
        pltpu.sync_copy(fps_hbm.at[wid], fidx_v)
        fbase = wid * (fch * 128)

        def fbody(g, _):
            pltpu.async_copy(u_hbm.at[fidx_v.at[g]], r0, s0).wait()
            pltpu.sync_copy(r0, c1_hbm.at[pl.ds(fbase + g * 128, 128)])
            return 0

        lax.fori_loop(0, fch, fbody, 0)

    return k(ttab, utab, nbr_idx, fps_idx)


MB = 512


def _ball_body(pts_ref, np_ref, tri128_ref, tri32_ref, out_ref):
    i = pl.program_id(0)
    b = i // (M // MB)
    pts = pts_ref[0]
    npts = np_ref[...]
    sqp = jnp.sum(pts * pts, axis=0, keepdims=True)
    sqn = jnp.sum(npts * npts, axis=1, keepdims=True)
    mm = jnp.dot(npts, pts, preferred_element_type=f32)
    d = sqn + sqp - 2.0 * mm
    mask = jnp.where(d <= R2, 1.0, 0.0).astype(jnp.bfloat16)
    maskr = mask.reshape(MB * 32, 128)
    loc = jnp.dot(maskr, tri128_ref[...], preferred_element_type=f32)
    cnt = loc[:, 127:128].reshape(MB, 32)
    e = jnp.dot(cnt.astype(jnp.bfloat16), tri32_ref[...],
                preferred_element_type=f32)
    s = e - cnt
    cg = loc.reshape(MB, 32, 128) + s[:, :, None]
    cgb = jnp.minimum(cg, 64.0).astype(jnp.bfloat16)
    one_b = jnp.ones((), jnp.bfloat16)
    zero_b = jnp.zeros((), jnp.bfloat16)
    cols = []
    for j in range(K):
        msk = jnp.where(cgb <= jnp.bfloat16(float(j)), one_b, zero_b)
        inner = jnp.sum(msk, axis=1)
        cols.append(jnp.sum(inner.astype(f32), axis=1, keepdims=True))
    acc = jnp.concatenate(cols, axis=1)
    acc = acc.astype(i32)
    first = jnp.where(acc[:, :1] >= N, 0, acc[:, :1])
    nbr = jnp.where(acc >= N, first, acc)
    out_ref[...] = nbr + b * N


def _ball(pts8, newpts):
    tri128 = jnp.triu(jnp.ones((128, 128), jnp.bfloat16))
    tri32 = jnp.triu(jnp.ones((32, 32), jnp.bfloat16))
    nblk = (B * M) // MB
    return pl.pallas_call(
        _ball_body,
        grid=(nblk,),
        in_specs=[
            pl.BlockSpec((1, 8, N), lambda i: (i // (M // MB), 0, 0)),
            pl.BlockSpec((MB, 8), lambda i: (i, 0)),
            pl.BlockSpec((128, 128), lambda i: (0, 0)),
            pl.BlockSpec((32, 32), lambda i: (0, 0)),
        ],
        out_specs=pl.BlockSpec((MB, K), lambda i: (i, 0)),
        out_shape=jax.ShapeDtypeStruct((B * M, K), i32),
    )(pts8, newpts, tri128, tri32)


BLK = 4096
NBLK = S // BLK
MROW = BLK // K


def _rep_c1(c1):
    return jnp.broadcast_to(c1[:, None, :], (MROW, K, 128)).reshape(BLK, 128)


def _stats_update(sums_ref, y, i, width):
    part = jnp.concatenate(
        [jnp.sum(y, axis=0, keepdims=True),
         jnp.sum(y * y, axis=0, keepdims=True)], axis=0)
    part = jnp.pad(part, ((0, 6), (0, 0)))

    @pl.when(i == 0)
    def _():
        sums_ref[...] = part

    @pl.when(i > 0)
    def _():
        sums_ref[...] = sums_ref[...] + part


def _scale_shift(aux_ref):
    sums = aux_ref[0:1, :]
    sumsq = aux_ref[1:2, :]
    g = aux_ref[2:3, :]
    bb = aux_ref[3:4, :]
    mean = sums * (1.0 / S)
    var = sumsq * (1.0 / S) - mean * mean
    rstd = lax.rsqrt(var + EPS)
    scale = g * rstd
    shift = bb - mean * scale
    return scale, shift


def _d1_body(y1r_ref, c1_ref, sums_ref):
    i = pl.program_id(0)
    y1 = y1r_ref[...] - _rep_c1(c1_ref[...])
    _stats_update(sums_ref, y1, i, 128)


def _d1(y1raw, c1):
    return pl.pallas_call(
        _d1_body,
        grid=(NBLK,),
        in_specs=[
            pl.BlockSpec((BLK, 128), lambda i: (i, 0)),
            pl.BlockSpec((MROW, 128), lambda i: (i, 0)),
        ],
        out_specs=pl.BlockSpec((8, 128), lambda i: (0, 0)),
        out_shape=jax.ShapeDtypeStruct((8, 128), f32),
    )(y1raw, c1)


def _d2_body(y1r_ref, c1_ref, aux_ref, w2_ref, y2_ref, sums_ref):
    i = pl.program_id(0)
    scale, shift = _scale_shift(aux_ref)
    y1 = y1r_ref[...] - _rep_c1(c1_ref[...])
    h1 = jnp.maximum(y1 * scale + shift, 0.0)
    y2 = jnp.dot(h1.astype(jnp.bfloat16), w2_ref[...].astype(jnp.bfloat16),
                 preferred_element_type=f32)
    y2_ref[...] = y2
    _stats_update(sums_ref, y2, i, 128)


def _d2(y1raw, c1, aux1, w2t):
    return pl.pallas_call(
        _d2_body,
        grid=(NBLK,),
        in_specs=[
            pl.BlockSpec((BLK, 128), lambda i: (i, 0)),
            pl.BlockSpec((MROW, 128), lambda i: (i, 0)),
            pl.BlockSpec((8, 128), lambda i: (0, 0)),
            pl.BlockSpec((128, 128), lambda i: (0, 0)),
        ],
        out_specs=[
            pl.BlockSpec((BLK, 128), lambda i: (i, 0)),
            pl.BlockSpec((8, 128), lambda i: (0, 0)),
        ],
        out_shape=[
            jax.ShapeDtypeStruct((S, 128), f32),
            jax.ShapeDtypeStruct((8, 128), f32),
        ],
    )(y1raw, c1, aux1, w2t)


def _d3_body(y2_ref, aux_ref, w3_ref, ymax_ref, sums_ref):
    i = pl.program_id(0)
    scale, shift = _scale_shift(aux_ref)
    h2 = jnp.maximum(y2_ref[...] * scale + shift, 0.0)
    y3 = jnp.dot(h2.astype(jnp.bfloat16), w3_ref[...].astype(jnp.bfloat16),
                 preferred_element_type=f32)
    ymax_ref[...] = jnp.max(y3.reshape(MROW, K, 256), axis=1)
    _stats_update(sums_ref, y3, i, 256)


def _d3(y2, aux2, w3t):
    return pl.pallas_call(
        _d3_body,
        grid=(NBLK,),
        in_specs=[
            pl.BlockSpec((BLK, 128), lambda i: (i, 0)),
            pl.BlockSpec((8, 128), lambda i: (0, 0)),
            pl.BlockSpec((128, 256), lambda i: (0, 0)),
        ],
        out_specs=[
            pl.BlockSpec((MROW, 256), lambda i: (i, 0)),
            pl.BlockSpec((8, 256), lambda i: (0, 0)),
        ],
        out_shape=[
            jax.ShapeDtypeStruct((B * M, 256), f32),
            jax.ShapeDtypeStruct((8, 256), f32),
        ],
    )(y2, aux2, w3t)


def _d4_body(ymax_ref, aux_ref, out_ref):
    scale, shift = _scale_shift(aux_ref)
    h = jnp.maximum(ymax_ref[...] * scale + shift, 0.0)
    out_ref[0] = h.T


def _d4(ymax, aux3):
    return pl.pallas_call(
        _d4_body,
        grid=(B,),
        in_specs=[
            pl.BlockSpec((M, 256), lambda i: (i, 0)),
            pl.BlockSpec((8, 256), lambda i: (0, 0)),
        ],
        out_specs=pl.BlockSpec((1, 256, M), lambda i: (i, 0, 0)),
        out_shape=jax.ShapeDtypeStruct((B, 256, M), f32),
    )(ymax, aux3)


def _aux(sums, g, b, width):
    return jnp.concatenate(
        [sums[:2], g.reshape(1, width), b.reshape(1, width),
         jnp.zeros((4, width), f32)], axis=0)


def kernel(xyz, feature, W1, g1, b1, W2, g2, b2, W3, g3, b3):
    fps_t = _fps(xyz)
    fps_idx = fps_t.T
    offs = (jnp.arange(B, dtype=i32) * N)[:, None]
    fpsflat = (fps_idx + offs).reshape(NW, (B * M) // NW // 128, 128)

    ptsT = jnp.transpose(xyz, (0, 2, 1))
    xyzpad = jnp.pad(ptsT, ((0, 0), (0, 0), (0, 125))).reshape(B * N, 128)
    newpts128 = _sc_gather_small(xyzpad, fpsflat)
    newpts = newpts128[:, :8]
    new_xyz = jnp.transpose(newpts[:, :3].reshape(B, M, 3), (0, 2, 1))

    W1a, W1b, W1c = W1[:, :3], W1[:, 3:3 + C], W1[:, 3 + C:]
    pts8 = jnp.pad(xyz, ((0, 0), (0, 5), (0, 0)))
    wtx = jnp.pad(W1a.T, ((0, 5), (0, 0)))
    ttab, utab = _proj(feature, pts8, (W1b + W1c).T, wtx, W1c.T)

    nbr = _ball(pts8, newpts)
    nbrflat = nbr.reshape(NW, (S // NW) // 128, 128)

    y1raw, c1 = _sc_gather_big(ttab, utab, nbrflat, fpsflat)

    sums1 = _d1(y1raw, c1)
    y2, sums2 = _d2(y1raw, c1, _aux(sums1, g1, b1, 128), W2.T)
    y3, sums3 = _d3(y2, _aux(sums2, g2, b2, 128), W3.T)
    new_feature = _d4(y3, _aux(sums3, g3, b3, 256))
    return new_xyz, new_feature


# speedup vs baseline: 1.1022x; 1.1022x over previous
import functools

import jax
import jax.numpy as jnp
from jax import lax
from jax.experimental import pallas as pl
from jax.experimental.pallas import tpu as pltpu
from jax.experimental.pallas import tpu_sc as plsc

B = 8
N = 4096
C = 128
M = 1024
K = 32
R2 = 0.2 * 0.2
EPS = 1e-5
S = B * M * K

NC, NS = 2, 16
NW = NC * NS

f32 = jnp.float32
i32 = jnp.int32


def _fps_body(xyz_ref, out_ref):
    X = xyz_ref[:, 0, :]
    Y = xyz_ref[:, 1, :]
    Z = xyz_ref[:, 2, :]
    iota = lax.broadcasted_iota(i32, (B, N), 1)

    def body(i, carry):
        dists, far = carry
        out_ref[pl.ds(i, 1), :] = far[None, :]
        oh = iota == far[:, None]
        sel = jnp.concatenate([jnp.where(oh, X, 0.0),
                               jnp.where(oh, Y, 0.0),
                               jnp.where(oh, Z, 0.0)], axis=0)
        c = jnp.sum(sel, axis=1, keepdims=True)
        dx = X - c[0:B]
        dy = Y - c[B:2 * B]
        dz = Z - c[2 * B:3 * B]
        d = (dx * dx + dy * dy) + dz * dz
        dists = jnp.minimum(dists, d)
        maxv = jnp.max(dists, axis=1, keepdims=True)
        far = jnp.min(jnp.where(dists == maxv, iota, N), axis=1).astype(i32)
        return dists, far

    dists0 = jnp.full((B, N), 1e10, f32)
    far0 = jnp.zeros((B,), i32)
    lax.fori_loop(0, M, body, (dists0, far0))


def _fps(xyz):
    return pl.pallas_call(
        _fps_body,
        out_shape=jax.ShapeDtypeStruct((M, B), i32),
    )(xyz)


def _proj_body(f_ref, p_ref, wtf_ref, wtx_ref, wuf_ref, t_ref, u_ref):
    f = f_ref[0]
    p = p_ref[0]
    dn = (((0,), (0,)), ((), ()))
    xp = lax.dot_general(p, wtx_ref[...], dn, preferred_element_type=f32)
    t_ref[...] = lax.dot_general(f, wtf_ref[...], dn,
                                 preferred_element_type=f32) + xp
    u_ref[...] = lax.dot_general(f, wuf_ref[...], dn,
                                 preferred_element_type=f32) + xp


def _proj(feature, pts8, wtf, wtx, wuf):
    return pl.pallas_call(
        _proj_body,
        grid=(B,),
        in_specs=[
            pl.BlockSpec((1, C, N), lambda i: (i, 0, 0)),
            pl.BlockSpec((1, 8, N), lambda i: (i, 0, 0)),
            pl.BlockSpec((C, 128), lambda i: (0, 0)),
            pl.BlockSpec((8, 128), lambda i: (0, 0)),
            pl.BlockSpec((C, 128), lambda i: (0, 0)),
        ],
        out_specs=[
            pl.BlockSpec((N, 128), lambda i: (i, 0)),
            pl.BlockSpec((N, 128), lambda i: (i, 0)),
        ],
        out_shape=[
            jax.ShapeDtypeStruct((B * N, 128), f32),
            jax.ShapeDtypeStruct((B * N, 128), f32),
        ],
    )(feature, pts8, wtf, wtx, wuf)


def _sc_gather_small(table, idx):
    chn = idx.shape[1]
    rows = chn * 128
    mesh = plsc.VectorSubcoreMesh(core_axis_name="c", subcore_axis_name="s")

    @functools.partial(
        pl.kernel,
        mesh=mesh,
        out_type=jax.ShapeDtypeStruct((NW * rows, 128), f32),
        scratch_types=[
            pltpu.VMEM((chn, 128), i32),
            pltpu.VMEM((128, 128), f32),
            pltpu.SemaphoreType.DMA,
        ],
    )
    def k(table_hbm, idx_hbm, out_hbm, idx_v, rows_v, sem):
        wid = lax.axis_index("s") * NC + lax.axis_index("c")
        base = wid * rows
        pltpu.sync_copy(idx_hbm.at[wid], idx_v)

        def body(g, _):
            pltpu.async_copy(table_hbm.at[idx_v.at[g]], rows_v, sem).wait()
            pltpu.sync_copy(rows_v, out_hbm.at[pl.ds(base + g * 128, 128)])
            return 0

        lax.fori_loop(0, chn, body, 0)

    return k(table, idx)


def _sc_gather_big(ttab, utab, nbr_idx, fps_idx):
    chn = nbr_idx.shape[1]
    fch = fps_idx.shape[1]
    mesh = plsc.VectorSubcoreMesh(core_axis_name="c", subcore_axis_name="s")

    @functools.partial(
        pl.kernel,
        mesh=mesh,
        out_type=(
            jax.ShapeDtypeStruct((S, 128), f32),
            jax.ShapeDtypeStruct((B * M, 128), f32),
        ),
        scratch_types=[
            pltpu.VMEM((chn, 128), i32),
            pltpu.VMEM((128, 128), f32),
            pltpu.VMEM((128, 128), f32),
            pltpu.VMEM((128, 128), f32),
            pltpu.VMEM((128, 128), f32),
            pltpu.VMEM((fch, 128), i32),
            pltpu.SemaphoreType.DMA,
            pltpu.SemaphoreType.DMA,
            pltpu.SemaphoreType.DMA,
            pltpu.SemaphoreType.DMA,
            pltpu.SemaphoreType.DMA,
        ],
    )
    def k(t_hbm, u_hbm, nbr_hbm, fps_hbm, y1_hbm, c1_hbm,
          idx_v, r0, r1, r2, r3, fidx_v, s0, s1, s2, s3, sw):
        wid = lax.axis_index("s") * NC + lax.axis_index("c")
        base = wid * (chn * 128)
        pltpu.sync_copy(nbr_hbm.at[wid], idx_v)
        bufs = (r0, r1, r2, r3)
        sems = (s0, s1, s2, s3)

        def body(g4, _):
            g = g4 * 4
            hs = [pltpu.async_copy(t_hbm.at[idx_v.at[g + q]], bufs[q],
                                   sems[q]) for q in range(4)]
            ws = []
            for q in range(4):
                hs[q].wait()
                ws.append(pltpu.async_copy(
                    bufs[q], y1_hbm.at[pl.ds(base + (g + q) * 128, 128)], sw))
            for w in ws:
                w.wait()
            return 0

        lax.fori_loop(0, chn // 4, body, 0)
        pltpu.sync_copy(fps_hbm.at[wid], fidx_v)
        fbase = wid * (fch * 128)

        def fbody(g, _):
            pltpu.async_copy(u_hbm.at[fidx_v.at[g]], r0, s0).wait()
            pltpu.sync_copy(r0, c1_hbm.at[pl.ds(fbase + g * 128, 128)])
            return 0

        lax.fori_loop(0, fch, fbody, 0)

    return k(ttab, utab, nbr_idx, fps_idx)


MB = 512


def _ball_body(pts_ref, np_ref, tri128_ref, tri32_ref, out_ref):
    i = pl.program_id(0)
    b = i // (M // MB)
    pts = pts_ref[0]
    npts = np_ref[...]
    sqp = jnp.sum(pts * pts, axis=0, keepdims=True)
    sqn = jnp.sum(npts * npts, axis=1, keepdims=True)
    mm = jnp.dot(npts, pts, preferred_element_type=f32)
    d = sqn + sqp - 2.0 * mm
    mask = jnp.where(d <= R2, 1.0, 0.0).astype(jnp.bfloat16)
    maskr = mask.reshape(MB * 32, 128)
    loc = jnp.dot(maskr, tri128_ref[...], preferred_element_type=f32)
    cnt = loc[:, 127:128].reshape(MB, 32)
    e = jnp.dot(cnt.astype(jnp.bfloat16), tri32_ref[...],
                preferred_element_type=f32)
    s = e - cnt
    cg = loc.reshape(MB, 32, 128) + s[:, :, None]
    cgf = cg.reshape(MB, N)
    cols = []
    for j in range(K):
        cols.append(jnp.sum(jnp.where(cgf <= float(j), 1.0, 0.0), axis=1,
                            keepdims=True))
    acc = jnp.concatenate(cols, axis=1)
    acc = acc.astype(i32)
    first = jnp.where(acc[:, :1] >= N, 0, acc[:, :1])
    nbr = jnp.where(acc >= N, first, acc)
    out_ref[...] = nbr + b * N


def _ball(pts8, newpts):
    tri128 = jnp.triu(jnp.ones((128, 128), jnp.bfloat16))
    tri32 = jnp.triu(jnp.ones((32, 32), jnp.bfloat16))
    nblk = (B * M) // MB
    return pl.pallas_call(
        _ball_body,
        grid=(nblk,),
        in_specs=[
            pl.BlockSpec((1, 8, N), lambda i: (i // (M // MB), 0, 0)),
            pl.BlockSpec((MB, 8), lambda i: (i, 0)),
            pl.BlockSpec((128, 128), lambda i: (0, 0)),
            pl.BlockSpec((32, 32), lambda i: (0, 0)),
        ],
        out_specs=pl.BlockSpec((MB, K), lambda i: (i, 0)),
        out_shape=jax.ShapeDtypeStruct((B * M, K), i32),
    )(pts8, newpts, tri128, tri32)


BLK = 4096
NBLK = S // BLK
MROW = BLK // K


def _rep_c1(c1):
    return jnp.broadcast_to(c1[:, None, :], (MROW, K, 128)).reshape(BLK, 128)


def _stats_update(sums_ref, y, i, width):
    part = jnp.concatenate(
        [jnp.sum(y, axis=0, keepdims=True),
         jnp.sum(y * y, axis=0, keepdims=True)], axis=0)
    part = jnp.pad(part, ((0, 6), (0, 0)))

    @pl.when(i == 0)
    def _():
        sums_ref[...] = part

    @pl.when(i > 0)
    def _():
        sums_ref[...] = sums_ref[...] + part


def _scale_shift(aux_ref):
    sums = aux_ref[0:1, :]
    sumsq = aux_ref[1:2, :]
    g = aux_ref[2:3, :]
    bb = aux_ref[3:4, :]
    mean = sums * (1.0 / S)
    var = sumsq * (1.0 / S) - mean * mean
    rstd = lax.rsqrt(var + EPS)
    scale = g * rstd
    shift = bb - mean * scale
    return scale, shift


def _d1_body(y1r_ref, c1_ref, sums_ref):
    i = pl.program_id(0)
    y1 = y1r_ref[...] - _rep_c1(c1_ref[...])
    _stats_update(sums_ref, y1, i, 128)


def _d1(y1raw, c1):
    return pl.pallas_call(
        _d1_body,
        grid=(NBLK,),
        in_specs=[
            pl.BlockSpec((BLK, 128), lambda i: (i, 0)),
            pl.BlockSpec((MROW, 128), lambda i: (i, 0)),
        ],
        out_specs=pl.BlockSpec((8, 128), lambda i: (0, 0)),
        out_shape=jax.ShapeDtypeStruct((8, 128), f32),
    )(y1raw, c1)


def _d2_body(y1r_ref, c1_ref, aux_ref, w2_ref, y2_ref, sums_ref):
    i = pl.program_id(0)
    scale, shift = _scale_shift(aux_ref)
    y1 = y1r_ref[...] - _rep_c1(c1_ref[...])
    h1 = jnp.maximum(y1 * scale + shift, 0.0)
    y2 = jnp.dot(h1.astype(jnp.bfloat16), w2_ref[...].astype(jnp.bfloat16),
                 preferred_element_type=f32)
    y2_ref[...] = y2
    _stats_update(sums_ref, y2, i, 128)


def _d2(y1raw, c1, aux1, w2t):
    return pl.pallas_call(
        _d2_body,
        grid=(NBLK,),
        in_specs=[
            pl.BlockSpec((BLK, 128), lambda i: (i, 0)),
            pl.BlockSpec((MROW, 128), lambda i: (i, 0)),
            pl.BlockSpec((8, 128), lambda i: (0, 0)),
            pl.BlockSpec((128, 128), lambda i: (0, 0)),
        ],
        out_specs=[
            pl.BlockSpec((BLK, 128), lambda i: (i, 0)),
            pl.BlockSpec((8, 128), lambda i: (0, 0)),
        ],
        out_shape=[
            jax.ShapeDtypeStruct((S, 128), f32),
            jax.ShapeDtypeStruct((8, 128), f32),
        ],
    )(y1raw, c1, aux1, w2t)


def _d3_body(y2_ref, aux_ref, w3_ref, ymax_ref, sums_ref):
    i = pl.program_id(0)
    scale, shift = _scale_shift(aux_ref)
    h2 = jnp.maximum(y2_ref[...] * scale + shift, 0.0)
    y3 = jnp.dot(h2.astype(jnp.bfloat16), w3_ref[...].astype(jnp.bfloat16),
                 preferred_element_type=f32)
    ymax_ref[...] = jnp.max(y3.reshape(MROW, K, 256), axis=1)
    _stats_update(sums_ref, y3, i, 256)


def _d3(y2, aux2, w3t):
    return pl.pallas_call(
        _d3_body,
        grid=(NBLK,),
        in_specs=[
            pl.BlockSpec((BLK, 128), lambda i: (i, 0)),
            pl.BlockSpec((8, 128), lambda i: (0, 0)),
            pl.BlockSpec((128, 256), lambda i: (0, 0)),
        ],
        out_specs=[
            pl.BlockSpec((MROW, 256), lambda i: (i, 0)),
            pl.BlockSpec((8, 256), lambda i: (0, 0)),
        ],
        out_shape=[
            jax.ShapeDtypeStruct((B * M, 256), f32),
            jax.ShapeDtypeStruct((8, 256), f32),
        ],
    )(y2, aux2, w3t)


def _d4_body(ymax_ref, aux_ref, out_ref):
    scale, shift = _scale_shift(aux_ref)
    h = jnp.maximum(ymax_ref[...] * scale + shift, 0.0)
    out_ref[0] = h.T


def _d4(ymax, aux3):
    return pl.pallas_call(
        _d4_body,
        grid=(B,),
        in_specs=[
            pl.BlockSpec((M, 256), lambda i: (i, 0)),
            pl.BlockSpec((8, 256), lambda i: (0, 0)),
        ],
        out_specs=pl.BlockSpec((1, 256, M), lambda i: (i, 0, 0)),
        out_shape=jax.ShapeDtypeStruct((B, 256, M), f32),
    )(ymax, aux3)


def _aux(sums, g, b, width):
    return jnp.concatenate(
        [sums[:2], g.reshape(1, width), b.reshape(1, width),
         jnp.zeros((4, width), f32)], axis=0)


def kernel(xyz, feature, W1, g1, b1, W2, g2, b2, W3, g3, b3):
    fps_t = _fps(xyz)
    fps_idx = fps_t.T
    offs = (jnp.arange(B, dtype=i32) * N)[:, None]
    fpsflat = (fps_idx + offs).reshape(NW, (B * M) // NW // 128, 128)

    ptsT = jnp.transpose(xyz, (0, 2, 1))
    xyzpad = jnp.pad(ptsT, ((0, 0), (0, 0), (0, 125))).reshape(B * N, 128)
    newpts128 = _sc_gather_small(xyzpad, fpsflat)
    newpts = newpts128[:, :8]
    new_xyz = jnp.transpose(newpts[:, :3].reshape(B, M, 3), (0, 2, 1))

    W1a, W1b, W1c = W1[:, :3], W1[:, 3:3 + C], W1[:, 3 + C:]
    pts8 = jnp.pad(xyz, ((0, 0), (0, 5), (0, 0)))
    wtx = jnp.pad(W1a.T, ((0, 5), (0, 0)))
    ttab, utab = _proj(feature, pts8, (W1b + W1c).T, wtx, W1c.T)

    nbr = _ball(pts8, newpts)
    nbrflat = nbr.reshape(NW, (S // NW) // 128, 128)

    y1raw, c1 = _sc_gather_big(ttab, utab, nbrflat, fpsflat)

    sums1 = _d1(y1raw, c1)
    y2, sums2 = _d2(y1raw, c1, _aux(sums1, g1, b1, 128), W2.T)
    y3, sums3 = _d3(y2, _aux(sums2, g2, b2, 128), W3.T)
    new_feature = _d4(y3, _aux(sums3, g3, b3, 256))
    return new_xyz, new_feature


# speedup vs baseline: 1.1936x; 1.0829x over previous
import functools

import jax
import jax.numpy as jnp
from jax import lax
from jax.experimental import pallas as pl
from jax.experimental.pallas import tpu as pltpu
from jax.experimental.pallas import tpu_sc as plsc

B = 8
N = 4096
C = 128
M = 1024
K = 32
R2 = 0.2 * 0.2
EPS = 1e-5
S = B * M * K

NC, NS = 2, 16
NW = NC * NS

f32 = jnp.float32
i32 = jnp.int32


def _fps_body(xyz_ref, out_ref):
    X = xyz_ref[:, 0, :]
    Y = xyz_ref[:, 1, :]
    Z = xyz_ref[:, 2, :]
    iota = lax.broadcasted_iota(i32, (B, N), 1)

    def body(i, carry):
        dists, far = carry
        out_ref[pl.ds(i, 1), :] = far[None, :]
        oh = iota == far[:, None]
        cx = jnp.sum(jnp.where(oh, X, 0.0), axis=1, keepdims=True)
        cy = jnp.sum(jnp.where(oh, Y, 0.0), axis=1, keepdims=True)
        cz = jnp.sum(jnp.where(oh, Z, 0.0), axis=1, keepdims=True)
        dx = X - cx
        dy = Y - cy
        dz = Z - cz
        d = (dx * dx + dy * dy) + dz * dz
        dists = jnp.minimum(dists, d)
        far = jnp.argmax(dists, axis=1).astype(i32)
        return dists, far

    dists0 = jnp.full((B, N), 1e10, f32)
    far0 = jnp.zeros((B,), i32)
    lax.fori_loop(0, M, body, (dists0, far0))


def _fps(xyz):
    return pl.pallas_call(
        _fps_body,
        out_shape=jax.ShapeDtypeStruct((M, B), i32),
    )(xyz)


def _proj_body(f_ref, p_ref, wtf_ref, wtx_ref, wuf_ref, t_ref, u_ref):
    f = f_ref[0]
    p = p_ref[0]
    dn = (((0,), (0,)), ((), ()))
    xp = lax.dot_general(p, wtx_ref[...], dn, preferred_element_type=f32)
    t_ref[...] = lax.dot_general(f, wtf_ref[...], dn,
                                 preferred_element_type=f32) + xp
    u_ref[...] = lax.dot_general(f, wuf_ref[...], dn,
                                 preferred_element_type=f32) + xp


def _proj(feature, pts8, wtf, wtx, wuf):
    return pl.pallas_call(
        _proj_body,
        grid=(B,),
        in_specs=[
            pl.BlockSpec((1, C, N), lambda i: (i, 0, 0)),
            pl.BlockSpec((1, 8, N), lambda i: (i, 0, 0)),
            pl.BlockSpec((C, 128), lambda i: (0, 0)),
            pl.BlockSpec((8, 128), lambda i: (0, 0)),
            pl.BlockSpec((C, 128), lambda i: (0, 0)),
        ],
        out_specs=[
            pl.BlockSpec((N, 128), lambda i: (i, 0)),
            pl.BlockSpec((N, 128), lambda i: (i, 0)),
        ],
        out_shape=[
            jax.ShapeDtypeStruct((B * N, 128), f32),
            jax.ShapeDtypeStruct((B * N, 128), f32),
        ],
    )(feature, pts8, wtf, wtx, wuf)


def _sc_gather_small(table, idx):
    chn = idx.shape[1]
    rows = chn * 128
    mesh = plsc.VectorSubcoreMesh(core_axis_name="c", subcore_axis_name="s")

    @functools.partial(
        pl.kernel,
        mesh=mesh,
        out_type=jax.ShapeDtypeStruct((NW * rows, 128), f32),
        scratch_types=[
            pltpu.VMEM((chn, 128), i32),
            pltpu.VMEM((128, 128), f32),
            pltpu.SemaphoreType.DMA,
        ],
    )
    def k(table_hbm, idx_hbm, out_hbm, idx_v, rows_v, sem):
        wid = lax.axis_index("s") * NC + lax.axis_index("c")
        base = wid * rows
        pltpu.sync_copy(idx_hbm.at[wid], idx_v)

        def body(g, _):
            pltpu.async_copy(table_hbm.at[idx_v.at[g]], rows_v, sem).wait()
            pltpu.sync_copy(rows_v, out_hbm.at[pl.ds(base + g * 128, 128)])
            return 0

        lax.fori_loop(0, chn, body, 0)

    return k(table, idx)


def _sc_gather_big(ttab, utab, nbr_idx, fps_idx):
    chn = nbr_idx.shape[1]
    fch = fps_idx.shape[1]
    mesh = plsc.VectorSubcoreMesh(core_axis_name="c", subcore_axis_name="s")

    @functools.partial(
        pl.kernel,
        mesh=mesh,
        out_type=(
            jax.ShapeDtypeStruct((S, 128), f32),
            jax.ShapeDtypeStruct((B * M, 128), f32),
        ),
        scratch_types=[
            pltpu.VMEM((chn, 128), i32),
            pltpu.VMEM((128, 128), f32),
            pltpu.VMEM((128, 128), f32),
            pltpu.VMEM((128, 128), f32),
            pltpu.VMEM((128, 128), f32),
            pltpu.VMEM((fch, 128), i32),
            pltpu.VMEM((128, 128), f32),
            pltpu.SemaphoreType.DMA,
            pltpu.SemaphoreType.DMA,
            pltpu.SemaphoreType.DMA,
            pltpu.SemaphoreType.DMA,
            pltpu.SemaphoreType.DMA,
        ],
    )
    def k(t_hbm, u_hbm, nbr_hbm, fps_hbm, y1_hbm, c1_hbm,
          idx_v, r0, r1, r2, r3, fidx_v, fr_v, s0, s1, s2, s3, sw):
        wid = lax.axis_index("s") * NC + lax.axis_index("c")
        base = wid * (chn * 128)
        pltpu.sync_copy(nbr_hbm.at[wid], idx_v)
        bufs = (r0, r1, r2, r3)
        sems = (s0, s1, s2, s3)

        def body(g4, _):
            g = g4 * 4
            hs = [pltpu.async_copy(t_hbm.at[idx_v.at[g + q]], bufs[q],
                                   sems[q]) for q in range(4)]
            ws = []
            for q in range(4):
                hs[q].wait()
                ws.append(pltpu.async_copy(
                    bufs[q], y1_hbm.at[pl.ds(base + (g + q) * 128, 128)], sw))
            for w in ws:
                w.wait()
            return 0

        lax.fori_loop(0, chn // 4, body, 0)
        pltpu.sync_copy(fps_hbm.at[wid], fidx_v)
        fbase = wid * (fch * 128)

        def fbody(g, _):
            pltpu.async_copy(u_hbm.at[fidx_v.at[g]], fr_v, s0).wait()
            pltpu.sync_copy(fr_v, c1_hbm.at[pl.ds(fbase + g * 128, 128)])
            return 0

        lax.fori_loop(0, fch, fbody, 0)

    return k(ttab, utab, nbr_idx, fps_idx)


MB = 512


def _ball_body(pts_ref, np_ref, tri128_ref, tri32_ref, out_ref):
    i = pl.program_id(0)
    b = i // (M // MB)
    pts = pts_ref[0]
    npts = np_ref[...]
    sqp = jnp.sum(pts * pts, axis=0, keepdims=True)
    sqn = jnp.sum(npts * npts, axis=1, keepdims=True)
    mm = jnp.dot(npts, pts, preferred_element_type=f32)
    d = sqn + sqp - 2.0 * mm
    mask = jnp.where(d <= R2, 1.0, 0.0).astype(jnp.bfloat16)
    maskr = mask.reshape(MB * 32, 128)
    loc = jnp.dot(maskr, tri128_ref[...], preferred_element_type=f32)
    cnt = loc[:, 127:128].reshape(MB, 32)
    e = jnp.dot(cnt.astype(jnp.bfloat16), tri32_ref[...],
                preferred_element_type=f32)
    s = e - cnt
    cg = loc.reshape(MB, 32, 128) + s[:, :, None]
    cgf = cg.reshape(MB, N)
    cols = []
    for j in range(K):
        cols.append(jnp.sum(jnp.where(cgf <= float(j), 1.0, 0.0), axis=1,
                            keepdims=True))
    acc = jnp.concatenate(cols, axis=1)
    acc = acc.astype(i32)
    first = jnp.where(acc[:, :1] >= N, 0, acc[:, :1])
    nbr = jnp.where(acc >= N, first, acc)
    out_ref[...] = nbr + b * N


def _ball(pts8, newpts):
    tri128 = jnp.triu(jnp.ones((128, 128), jnp.bfloat16))
    tri32 = jnp.triu(jnp.ones((32, 32), jnp.bfloat16))
    nblk = (B * M) // MB
    return pl.pallas_call(
        _ball_body,
        grid=(nblk,),
        in_specs=[
            pl.BlockSpec((1, 8, N), lambda i: (i // (M // MB), 0, 0)),
            pl.BlockSpec((MB, 8), lambda i: (i, 0)),
            pl.BlockSpec((128, 128), lambda i: (0, 0)),
            pl.BlockSpec((32, 32), lambda i: (0, 0)),
        ],
        out_specs=pl.BlockSpec((MB, K), lambda i: (i, 0)),
        out_shape=jax.ShapeDtypeStruct((B * M, K), i32),
    )(pts8, newpts, tri128, tri32)


BLK = 4096
NBLK = S // BLK
MROW = BLK // K


def _rep_c1(c1):
    return jnp.broadcast_to(c1[:, None, :], (MROW, K, 128)).reshape(BLK, 128)


def _stats_update(sums_ref, y, i, width):
    part = jnp.concatenate(
        [jnp.sum(y, axis=0, keepdims=True),
         jnp.sum(y * y, axis=0, keepdims=True)], axis=0)
    part = jnp.pad(part, ((0, 6), (0, 0)))

    @pl.when(i == 0)
    def _():
        sums_ref[...] = part

    @pl.when(i > 0)
    def _():
        sums_ref[...] = sums_ref[...] + part


def _scale_shift(aux_ref):
    sums = aux_ref[0:1, :]
    sumsq = aux_ref[1:2, :]
    g = aux_ref[2:3, :]
    bb = aux_ref[3:4, :]
    mean = sums * (1.0 / S)
    var = sumsq * (1.0 / S) - mean * mean
    rstd = lax.rsqrt(var + EPS)
    scale = g * rstd
    shift = bb - mean * scale
    return scale, shift


def _d1_body(y1r_ref, c1_ref, sums_ref):
    i = pl.program_id(0)
    y1 = y1r_ref[...] - _rep_c1(c1_ref[...])
    _stats_update(sums_ref, y1, i, 128)


def _d1(y1raw, c1):
    return pl.pallas_call(
        _d1_body,
        grid=(NBLK,),
        in_specs=[
            pl.BlockSpec((BLK, 128), lambda i: (i, 0)),
            pl.BlockSpec((MROW, 128), lambda i: (i, 0)),
        ],
        out_specs=pl.BlockSpec((8, 128), lambda i: (0, 0)),
        out_shape=jax.ShapeDtypeStruct((8, 128), f32),
    )(y1raw, c1)


def _d2_body(y1r_ref, c1_ref, aux_ref, w2_ref, y2_ref, sums_ref):
    i = pl.program_id(0)
    scale, shift = _scale_shift(aux_ref)
    y1 = y1r_ref[...] - _rep_c1(c1_ref[...])
    h1 = jnp.maximum(y1 * scale + shift, 0.0)
    y2 = jnp.dot(h1.astype(jnp.bfloat16), w2_ref[...].astype(jnp.bfloat16),
                 preferred_element_type=f32)
    y2_ref[...] = y2
    _stats_update(sums_ref, y2, i, 128)


def _d2(y1raw, c1, aux1, w2t):
    return pl.pallas_call(
        _d2_body,
        grid=(NBLK,),
        in_specs=[
            pl.BlockSpec((BLK, 128), lambda i: (i, 0)),
            pl.BlockSpec((MROW, 128), lambda i: (i, 0)),
            pl.BlockSpec((8, 128), lambda i: (0, 0)),
            pl.BlockSpec((128, 128), lambda i: (0, 0)),
        ],
        out_specs=[
            pl.BlockSpec((BLK, 128), lambda i: (i, 0)),
            pl.BlockSpec((8, 128), lambda i: (0, 0)),
        ],
        out_shape=[
            jax.ShapeDtypeStruct((S, 128), f32),
            jax.ShapeDtypeStruct((8, 128), f32),
        ],
    )(y1raw, c1, aux1, w2t)


def _d3_body(y2_ref, aux_ref, w3_ref, ymax_ref, sums_ref):
    i = pl.program_id(0)
    scale, shift = _scale_shift(aux_ref)
    h2 = jnp.maximum(y2_ref[...] * scale + shift, 0.0)
    y3 = jnp.dot(h2.astype(jnp.bfloat16), w3_ref[...].astype(jnp.bfloat16),
                 preferred_element_type=f32)
    ymax_ref[...] = jnp.max(y3.reshape(MROW, K, 256), axis=1)
    _stats_update(sums_ref, y3, i, 256)


def _d3(y2, aux2, w3t):
    return pl.pallas_call(
        _d3_body,
        grid=(NBLK,),
        in_specs=[
            pl.BlockSpec((BLK, 128), lambda i: (i, 0)),
            pl.BlockSpec((8, 128), lambda i: (0, 0)),
            pl.BlockSpec((128, 256), lambda i: (0, 0)),
        ],
        out_specs=[
            pl.BlockSpec((MROW, 256), lambda i: (i, 0)),
            pl.BlockSpec((8, 256), lambda i: (0, 0)),
        ],
        out_shape=[
            jax.ShapeDtypeStruct((B * M, 256), f32),
            jax.ShapeDtypeStruct((8, 256), f32),
        ],
    )(y2, aux2, w3t)


def _d4_body(ymax_ref, aux_ref, out_ref):
    scale, shift = _scale_shift(aux_ref)
    h = jnp.maximum(ymax_ref[...] * scale + shift, 0.0)
    out_ref[0] = h.T


def _d4(ymax, aux3):
    return pl.pallas_call(
        _d4_body,
        grid=(B,),
        in_specs=[
            pl.BlockSpec((M, 256), lambda i: (i, 0)),
            pl.BlockSpec((8, 256), lambda i: (0, 0)),
        ],
        out_specs=pl.BlockSpec((1, 256, M), lambda i: (i, 0, 0)),
        out_shape=jax.ShapeDtypeStruct((B, 256, M), f32),
    )(ymax, aux3)


def _aux(sums, g, b, width):
    return jnp.concatenate(
        [sums[:2], g.reshape(1, width), b.reshape(1, width),
         jnp.zeros((4, width), f32)], axis=0)


def kernel(xyz, feature, W1, g1, b1, W2, g2, b2, W3, g3, b3):
    fps_t = _fps(xyz)
    fps_idx = fps_t.T
    offs = (jnp.arange(B, dtype=i32) * N)[:, None]
    fpsflat = (fps_idx + offs).reshape(NW, (B * M) // NW // 128, 128)

    ptsT = jnp.transpose(xyz, (0, 2, 1))
    xyzpad = jnp.pad(ptsT, ((0, 0), (0, 0), (0, 125))).reshape(B * N, 128)
    newpts128 = _sc_gather_small(xyzpad, fpsflat)
    newpts = newpts128[:, :8]
    new_xyz = jnp.transpose(newpts[:, :3].reshape(B, M, 3), (0, 2, 1))

    W1a, W1b, W1c = W1[:, :3], W1[:, 3:3 + C], W1[:, 3 + C:]
    pts8 = jnp.pad(xyz, ((0, 0), (0, 5), (0, 0)))
    wtx = jnp.pad(W1a.T, ((0, 5), (0, 0)))
    ttab, utab = _proj(feature, pts8, (W1b + W1c).T, wtx, W1c.T)

    nbr = _ball(pts8, newpts)
    nbrflat = nbr.reshape(NW, (S // NW) // 128, 128)

    y1raw, c1 = _sc_gather_big(ttab, utab, nbrflat, fpsflat)

    sums1 = _d1(y1raw, c1)
    y2, sums2 = _d2(y1raw, c1, _aux(sums1, g1, b1, 128), W2.T)
    y3, sums3 = _d3(y2, _aux(sums2, g2, b2, 128), W3.T)
    new_feature = _d4(y3, _aux(sums3, g3, b3, 256))
    return new_xyz, new_feature
